# baseline (speedup 1.0000x reference)
import functools

import jax
import jax.numpy as jnp
from jax import lax
from jax.experimental import pallas as pl
from jax.experimental.pallas import tpu as pltpu
from jax.experimental.pallas import tpu_sc as plsc

N = 10000
E = 160000
T = 4
D_IN = 128
HEADS1 = 8
HID = 16
OUT = 64
SLOPE = 0.2

NC = 2
NS = 16
NWK = NC * NS
CH = 128
EP = 163840
EPG = EP // NS
K0 = 32
K1 = 48
NP = N
ROWS_PER_TILE = 624

DROW = 80
WACC = 64

BN = 1000


def _tables_body(x_ref, p_ref, z_ref):
    z_ref[0] = jnp.dot(x_ref[...], p_ref[0], preferred_element_type=jnp.float32)


def _build_tables(x, P):
    T_, K, Do = P.shape
    return pl.pallas_call(
        _tables_body,
        grid=(T_, N // BN),
        in_specs=[
            pl.BlockSpec((BN, K), lambda t, i: (i, 0)),
            pl.BlockSpec((1, K, Do), lambda t, i: (t, 0, 0)),
        ],
        out_specs=pl.BlockSpec((1, BN, Do), lambda t, i: (t, i, 0)),
        out_shape=jax.ShapeDtypeStruct((T_, N, Do), jnp.float32),
    )(x, P)


def _layer2_body(pa_ref, pb_ref, p2_ref, rw_ref, rb_ref, z_ref, r_ref, h_ref):
    t = pl.program_id(1)

    @pl.when(t == 0)
    def _():
        xa = pa_ref[0] + pa_ref[1]
        xb = pb_ref[0] + pb_ref[1]
        h_ref[:, :WACC] = jnp.where(xa > 0, xa, jnp.exp(xa) - 1.0)
        h_ref[:, WACC:] = jnp.where(xb > 0, xb, jnp.exp(xb) - 1.0)
        r_ref[...] = (
            jnp.dot(h_ref[...], rw_ref[...], preferred_element_type=jnp.float32)
            + rb_ref[...]
        )

    z_ref[0] = jnp.dot(h_ref[...], p2_ref[0], preferred_element_type=jnp.float32)


def _layer2_tables(pa, pb, P2, rw, rb):
    return pl.pallas_call(
        _layer2_body,
        grid=(N // BN, T),
        in_specs=[
            pl.BlockSpec((2, BN, WACC), lambda i, t: (0, i, 0)),
            pl.BlockSpec((2, BN, WACC), lambda i, t: (0, i, 0)),
            pl.BlockSpec((1, D_IN, DROW), lambda i, t: (t, 0, 0)),
            pl.BlockSpec((D_IN, OUT), lambda i, t: (0, 0)),
            pl.BlockSpec((1, OUT), lambda i, t: (0, 0)),
        ],
        out_specs=[
            pl.BlockSpec((1, BN, DROW), lambda i, t: (t, i, 0)),
            pl.BlockSpec((BN, OUT), lambda i, t: (i, 0)),
        ],
        out_shape=[
            jax.ShapeDtypeStruct((T, N, DROW), jnp.float32),
            jax.ShapeDtypeStruct((N, OUT), jnp.float32),
        ],
        scratch_shapes=[pltpu.VMEM((BN, D_IN), jnp.float32)],
    )(pa, pb, P2, rw, rb)


def _final_body(q_ref, r_ref, o_ref):
    o_ref[...] = q_ref[0] + q_ref[1] + r_ref[...]


def _final_combine(q, R):
    return pl.pallas_call(
        _final_body,
        grid=(N // BN,),
        in_specs=[
            pl.BlockSpec((2, BN, OUT), lambda i: (0, i, 0)),
            pl.BlockSpec((BN, OUT), lambda i: (i, 0)),
        ],
        out_specs=pl.BlockSpec((BN, OUT), lambda i: (i, 0)),
        out_shape=jax.ShapeDtypeStruct((N, OUT), jnp.float32),
    )(q, R)


def _make_sc_edge_kernel(D, W, NH, CHK=CH):
    mesh = plsc.VectorSubcoreMesh(core_axis_name="c", subcore_axis_name="s")

    @functools.partial(
        pl.kernel,
        mesh=mesh,
        out_type=jax.ShapeDtypeStruct((NC * NP, W), jnp.float32),
        compiler_params=pltpu.CompilerParams(use_tc_tiling_on_sc=False),
        scratch_types=[
            pltpu.VMEM((CHK,), jnp.int32),
            pltpu.VMEM((CHK,), jnp.int32),
            pltpu.VMEM((CHK,), jnp.int32),
            pltpu.VMEM((CHK,), jnp.float32),
            pltpu.VMEM((CHK, D), jnp.float32),
            pltpu.VMEM((CHK, W), jnp.float32),
            pltpu.VMEM((CHK, W), jnp.float32),
            pltpu.VMEM_SHARED((NP, W), jnp.float32),
            pltpu.SemaphoreType.DMA,
            pltpu.SemaphoreType.DMA,
            pltpu.SemaphoreType.DMA,
        ],
    )
    def k(table_h, si_h, dst_h, ew_h, zero_h, out_h,
          si_v, dst_a, dst_b, ew_v, rows_v, al_a, al_b, acc_sh,
          gsem, ssem_a, ssem_b):
        cid = lax.axis_index("c")
        sid = lax.axis_index("s")
        wid = sid * NC + cid
        r0 = pl.multiple_of(sid * ROWS_PER_TILE, 8)
        pltpu.sync_copy(zero_h.at[pl.ds(r0, ROWS_PER_TILE)],
                        acc_sh.at[pl.ds(r0, ROWS_PER_TILE)])

        @pl.when(sid == NS - 1)
        def _():
            pltpu.sync_copy(zero_h.at[pl.ds(NS * ROWS_PER_TILE, NP - NS * ROWS_PER_TILE)],
                            acc_sh.at[pl.ds(NS * ROWS_PER_TILE, NP - NS * ROWS_PER_TILE)])

        plsc.subcore_barrier()

        nch = jnp.where(cid == 0, K0, K1)
        base0 = sid * EPG + jnp.where(cid == 0, 0, K0 * CHK)

        def process(c, dst_v, al_v, ssem):
            base = pl.multiple_of(base0 + c * CHK, 8)
            pltpu.sync_copy(si_h.at[pl.ds(base, CHK)], si_v)
            pltpu.async_copy(table_h.at[si_v], rows_v, gsem)

            @pl.when(c >= 2)
            def _():
                pltpu.make_async_copy(al_v, acc_sh.at[dst_v], ssem).wait()

            pltpu.sync_copy(dst_h.at[pl.ds(base, CHK)], dst_v)
            pltpu.sync_copy(ew_h.at[pl.ds(base, CHK)], ew_v)
            pltpu.make_async_copy(table_h.at[si_v], rows_v, gsem).wait()

            def group(g, carry2):
                ew16 = ew_v[pl.ds(g * 16, 16)]
                for j in range(16):
                    e = g * 16 + j
                    lv = rows_v[e, pl.ds(W, 16)]
                    lv = jnp.where(lv >= 0, lv, SLOPE * lv)
                    att = (1.0 / (1.0 + jnp.exp(-lv))) * ew16[j]
                    for v in range(W // 16):
                        hk = (v * NH * 16) // W
                        al_v[e, pl.ds(v * 16, 16)] = (
                            rows_v[e, pl.ds(v * 16, 16)] * att[hk]
                        )
                return carry2

            lax.fori_loop(0, CHK // 16, group, 0)
            pltpu.async_copy(al_v, acc_sh.at[dst_v], ssem, add=True)

        def pair(p, carry):
            process(2 * p, dst_a, al_a, ssem_a)
            process(2 * p + 1, dst_b, al_b, ssem_b)
            return carry

        lax.fori_loop(0, nch // 2, pair, 0)
        pltpu.make_async_copy(al_a, acc_sh.at[dst_a], ssem_a).wait()
        pltpu.make_async_copy(al_b, acc_sh.at[dst_b], ssem_b).wait()
        plsc.subcore_barrier()
        pltpu.sync_copy(acc_sh.at[pl.ds(r0, ROWS_PER_TILE)],
                        out_h.at[pl.ds(cid * NP + r0, ROWS_PER_TILE)])

        @pl.when(sid == NS - 1)
        def _():
            pltpu.sync_copy(
                acc_sh.at[pl.ds(NS * ROWS_PER_TILE, NP - NS * ROWS_PER_TILE)],
                out_h.at[pl.ds(cid * NP + NS * ROWS_PER_TILE,
                               NP - NS * ROWS_PER_TILE)])

    return k


_sc_edge = _make_sc_edge_kernel(DROW, WACC, 4)


def kernel(feat, edge_index, edge_weight, ntype_idxs, etype_idxs,
           W1, a_l1, a_r1, W2, a_l2, a_r2, res_W2, res_b2):
    src = edge_index[0]
    dst = edge_index[1]

    a1 = (a_l1 + a_r1).reshape(T, D_IN, HEADS1, HID).sum(-1)
    C1 = jnp.matmul(W1, a1)
    zpad = jnp.zeros((T, D_IN, DROW - WACC - 4), jnp.float32)
    P1A = jnp.concatenate([W1[:, :, :WACC], C1[:, :, :4], zpad], axis=2)
    P1B = jnp.concatenate([W1[:, :, WACC:], C1[:, :, 4:], zpad], axis=2)

    idxc = jnp.arange(D_IN)
    perm = (idxc % HID) * HEADS1 + idxc // HID
    W2p = W2[:, perm, :]
    rwp = res_W2[perm, :]
    a2 = (a_l2 + a_r2).sum(axis=2)
    C2 = jnp.einsum('tko,to->tk', W2p, a2)
    P2 = jnp.concatenate(
        [W2p, jnp.repeat(C2[:, :, None], 4, axis=2), zpad], axis=2)

    pad = EP - E
    si = jnp.concatenate([etype_idxs * N + src,
                          jnp.zeros((pad,), jnp.int32)])
    dstp = jnp.concatenate([dst, jnp.zeros((pad,), jnp.int32)])
    ewp = jnp.concatenate([edge_weight, jnp.zeros((pad,), jnp.float32)])

    Z1A = _build_tables(feat, P1A).reshape(T * N, DROW)
    Z1B = _build_tables(feat, P1B).reshape(T * N, DROW)
    zeros = jnp.zeros((NP, WACC), jnp.float32)
    pa = _sc_edge(Z1A, si, dstp, ewp, zeros).reshape(NC, NP, WACC)
    pb = _sc_edge(Z1B, si, dstp, ewp, zeros).reshape(NC, NP, WACC)

    Z2_R = _layer2_tables(pa, pb, P2, rwp, res_b2.reshape(1, OUT))
    Z2 = Z2_R[0].reshape(T * N, DROW)
    R = Z2_R[1]

    q = _sc_edge(Z2, si, dstp, ewp, zeros).reshape(NC, NP, OUT)
    return _final_combine(q, R)


# speedup vs baseline: 1.2342x; 1.2342x over previous
import functools

import jax
import jax.numpy as jnp
from jax import lax
from jax.experimental import pallas as pl
from jax.experimental.pallas import tpu as pltpu
from jax.experimental.pallas import tpu_sc as plsc

N = 10000
E = 160000
T = 4
D_IN = 128
HEADS1 = 8
HID = 16
OUT = 64
SLOPE = 0.2

NC = 2
NS = 16
NWK = NC * NS
CH = 128
EP = 163840
EPG = EP // NS
K0 = 48
K1 = 32
NP = N
ROWS_PER_TILE = 624

DROW = 80
WACC = 64

BN = 1000


def _tables_body(x_ref, p_ref, z_ref):
    z_ref[0] = jnp.dot(x_ref[...], p_ref[0], preferred_element_type=jnp.float32)


def _build_tables(x, P):
    T_, K, Do = P.shape
    return pl.pallas_call(
        _tables_body,
        grid=(T_, N // BN),
        in_specs=[
            pl.BlockSpec((BN, K), lambda t, i: (i, 0)),
            pl.BlockSpec((1, K, Do), lambda t, i: (t, 0, 0)),
        ],
        out_specs=pl.BlockSpec((1, BN, Do), lambda t, i: (t, i, 0)),
        out_shape=jax.ShapeDtypeStruct((T_, N, Do), jnp.float32),
    )(x, P)


def _layer2_body(pa_ref, pb_ref, p2_ref, rw_ref, rb_ref, z_ref, r_ref, h_ref):
    t = pl.program_id(1)

    @pl.when(t == 0)
    def _():
        xa = pa_ref[0] + pa_ref[1]
        xb = pb_ref[0] + pb_ref[1]
        h_ref[:, :WACC] = jnp.where(xa > 0, xa, jnp.exp(xa) - 1.0)
        h_ref[:, WACC:] = jnp.where(xb > 0, xb, jnp.exp(xb) - 1.0)
        r_ref[...] = (
            jnp.dot(h_ref[...], rw_ref[...], preferred_element_type=jnp.float32)
            + rb_ref[...]
        )

    z_ref[0] = jnp.dot(h_ref[...], p2_ref[0], preferred_element_type=jnp.float32)


def _layer2_tables(pa, pb, P2, rw, rb):
    return pl.pallas_call(
        _layer2_body,
        grid=(N // BN, T),
        in_specs=[
            pl.BlockSpec((2, BN, WACC), lambda i, t: (0, i, 0)),
            pl.BlockSpec((2, BN, WACC), lambda i, t: (0, i, 0)),
            pl.BlockSpec((1, D_IN, DROW), lambda i, t: (t, 0, 0)),
            pl.BlockSpec((D_IN, OUT), lambda i, t: (0, 0)),
            pl.BlockSpec((1, OUT), lambda i, t: (0, 0)),
        ],
        out_specs=[
            pl.BlockSpec((1, BN, DROW), lambda i, t: (t, i, 0)),
            pl.BlockSpec((BN, OUT), lambda i, t: (i, 0)),
        ],
        out_shape=[
            jax.ShapeDtypeStruct((T, N, DROW), jnp.float32),
            jax.ShapeDtypeStruct((N, OUT), jnp.float32),
        ],
        scratch_shapes=[pltpu.VMEM((BN, D_IN), jnp.float32)],
    )(pa, pb, P2, rw, rb)


def _final_body(q_ref, r_ref, o_ref):
    o_ref[...] = q_ref[0] + q_ref[1] + r_ref[...]


def _final_combine(q, R):
    return pl.pallas_call(
        _final_body,
        grid=(N // BN,),
        in_specs=[
            pl.BlockSpec((2, BN, OUT), lambda i: (0, i, 0)),
            pl.BlockSpec((BN, OUT), lambda i: (i, 0)),
        ],
        out_specs=pl.BlockSpec((BN, OUT), lambda i: (i, 0)),
        out_shape=jax.ShapeDtypeStruct((N, OUT), jnp.float32),
    )(q, R)


def _make_sc_edge_kernel(D, W, NH, CHK=CH):
    mesh = plsc.VectorSubcoreMesh(core_axis_name="c", subcore_axis_name="s")

    @functools.partial(
        pl.kernel,
        mesh=mesh,
        out_type=jax.ShapeDtypeStruct((NC * NP, W), jnp.float32),
        compiler_params=pltpu.CompilerParams(use_tc_tiling_on_sc=False),
        scratch_types=[
            pltpu.VMEM((CHK,), jnp.int32),
            pltpu.VMEM((CHK,), jnp.int32),
            pltpu.VMEM((CHK,), jnp.int32),
            pltpu.VMEM((CHK,), jnp.float32),
            pltpu.VMEM((CHK, D), jnp.float32),
            pltpu.VMEM((CHK, W), jnp.float32),
            pltpu.VMEM((CHK, W), jnp.float32),
            pltpu.VMEM_SHARED((NP, W), jnp.float32),
            pltpu.SemaphoreType.DMA,
            pltpu.SemaphoreType.DMA,
            pltpu.SemaphoreType.DMA,
        ],
    )
    def k(table_h, si_h, dst_h, ew_h, zero_h, out_h,
          si_v, dst_a, dst_b, ew_v, rows_v, al_a, al_b, acc_sh,
          gsem, ssem_a, ssem_b):
        cid = lax.axis_index("c")
        sid = lax.axis_index("s")
        wid = sid * NC + cid
        r0 = pl.multiple_of(sid * ROWS_PER_TILE, 8)
        pltpu.sync_copy(zero_h.at[pl.ds(r0, ROWS_PER_TILE)],
                        acc_sh.at[pl.ds(r0, ROWS_PER_TILE)])

        @pl.when(sid == NS - 1)
        def _():
            pltpu.sync_copy(zero_h.at[pl.ds(NS * ROWS_PER_TILE, NP - NS * ROWS_PER_TILE)],
                            acc_sh.at[pl.ds(NS * ROWS_PER_TILE, NP - NS * ROWS_PER_TILE)])

        plsc.subcore_barrier()

        nch = jnp.where(cid == 0, K0, K1)
        base0 = sid * EPG + jnp.where(cid == 0, 0, K0 * CHK)

        def process(c, dst_v, al_v, ssem):
            base = pl.multiple_of(base0 + c * CHK, 8)
            pltpu.sync_copy(si_h.at[pl.ds(base, CHK)], si_v)
            pltpu.async_copy(table_h.at[si_v], rows_v, gsem)

            @pl.when(c >= 2)
            def _():
                pltpu.make_async_copy(al_v, acc_sh.at[dst_v], ssem).wait()

            pltpu.sync_copy(dst_h.at[pl.ds(base, CHK)], dst_v)
            pltpu.sync_copy(ew_h.at[pl.ds(base, CHK)], ew_v)
            pltpu.make_async_copy(table_h.at[si_v], rows_v, gsem).wait()

            def group(g, carry2):
                ew16 = ew_v[pl.ds(g * 16, 16)]
                for j in range(16):
                    e = g * 16 + j
                    lv = rows_v[e, pl.ds(W, 16)]
                    lv = jnp.where(lv >= 0, lv, SLOPE * lv)
                    att = (1.0 / (1.0 + jnp.exp(-lv))) * ew16[j]
                    for v in range(W // 16):
                        hk = (v * NH * 16) // W
                        al_v[e, pl.ds(v * 16, 16)] = (
                            rows_v[e, pl.ds(v * 16, 16)] * att[hk]
                        )
                return carry2

            lax.fori_loop(0, CHK // 16, group, 0)
            pltpu.async_copy(al_v, acc_sh.at[dst_v], ssem, add=True)

        def pair(p, carry):
            process(2 * p, dst_a, al_a, ssem_a)
            process(2 * p + 1, dst_b, al_b, ssem_b)
            return carry

        lax.fori_loop(0, nch // 2, pair, 0)
        pltpu.make_async_copy(al_a, acc_sh.at[dst_a], ssem_a).wait()
        pltpu.make_async_copy(al_b, acc_sh.at[dst_b], ssem_b).wait()
        plsc.subcore_barrier()
        pltpu.sync_copy(acc_sh.at[pl.ds(r0, ROWS_PER_TILE)],
                        out_h.at[pl.ds(cid * NP + r0, ROWS_PER_TILE)])

        @pl.when(sid == NS - 1)
        def _():
            pltpu.sync_copy(
                acc_sh.at[pl.ds(NS * ROWS_PER_TILE, NP - NS * ROWS_PER_TILE)],
                out_h.at[pl.ds(cid * NP + NS * ROWS_PER_TILE,
                               NP - NS * ROWS_PER_TILE)])

    return k


_sc_edge = _make_sc_edge_kernel(DROW, WACC, 4)


def kernel(feat, edge_index, edge_weight, ntype_idxs, etype_idxs,
           W1, a_l1, a_r1, W2, a_l2, a_r2, res_W2, res_b2):
    src = edge_index[0]
    dst = edge_index[1]

    a1 = (a_l1 + a_r1).reshape(T, D_IN, HEADS1, HID).sum(-1)
    C1 = jnp.matmul(W1, a1)
    zpad = jnp.zeros((T, D_IN, DROW - WACC - 4), jnp.float32)
    P1A = jnp.concatenate([W1[:, :, :WACC], C1[:, :, :4], zpad], axis=2)
    P1B = jnp.concatenate([W1[:, :, WACC:], C1[:, :, 4:], zpad], axis=2)

    idxc = jnp.arange(D_IN)
    perm = (idxc % HID) * HEADS1 + idxc // HID
    W2p = W2[:, perm, :]
    rwp = res_W2[perm, :]
    a2 = (a_l2 + a_r2).sum(axis=2)
    C2 = jnp.einsum('tko,to->tk', W2p, a2)
    P2 = jnp.concatenate(
        [W2p, jnp.repeat(C2[:, :, None], 4, axis=2), zpad], axis=2)

    pad = EP - E
    si = jnp.concatenate([etype_idxs * N + src,
                          jnp.zeros((pad,), jnp.int32)])
    dstp = jnp.concatenate([dst, jnp.zeros((pad,), jnp.int32)])
    ewp = jnp.concatenate([edge_weight, jnp.zeros((pad,), jnp.float32)])

    Z1A = _build_tables(feat, P1A).reshape(T * N, DROW)
    Z1B = _build_tables(feat, P1B).reshape(T * N, DROW)
    zeros = jnp.zeros((NP, WACC), jnp.float32)
    pa = _sc_edge(Z1A, si, dstp, ewp, zeros).reshape(NC, NP, WACC)
    pb = _sc_edge(Z1B, si, dstp, ewp, zeros).reshape(NC, NP, WACC)

    Z2_R = _layer2_tables(pa, pb, P2, rwp, res_b2.reshape(1, OUT))
    Z2 = Z2_R[0].reshape(T * N, DROW)
    R = Z2_R[1]

    q = _sc_edge(Z2, si, dstp, ewp, zeros).reshape(NC, NP, OUT)
    return _final_combine(q, R)


# speedup vs baseline: 1.6931x; 1.3718x over previous
import functools

import jax
import jax.numpy as jnp
from jax import lax
from jax.experimental import pallas as pl
from jax.experimental.pallas import tpu as pltpu
from jax.experimental.pallas import tpu_sc as plsc

N = 10000
E = 160000
T = 4
D_IN = 128
HEADS1 = 8
HID = 16
OUT = 64
SLOPE = 0.2

NC = 2
NS = 16
NWK = NC * NS
CH = 128
EP = 163840
EPG = EP // NS
K0 = 48
K1 = 32
KMAX = 48
EPALLOC = EP + (KMAX - K1) * CH
NP = N
ROWS_PER_TILE = 624

DROW = 80
WACC = 64

BN = 1000


def _tables_body(x_ref, p_ref, z_ref):
    z_ref[0] = jnp.dot(x_ref[...], p_ref[0], preferred_element_type=jnp.float32)


def _build_tables(x, P):
    T_, K, Do = P.shape
    return pl.pallas_call(
        _tables_body,
        grid=(T_, N // BN),
        in_specs=[
            pl.BlockSpec((BN, K), lambda t, i: (i, 0)),
            pl.BlockSpec((1, K, Do), lambda t, i: (t, 0, 0)),
        ],
        out_specs=pl.BlockSpec((1, BN, Do), lambda t, i: (t, i, 0)),
        out_shape=jax.ShapeDtypeStruct((T_, N, Do), jnp.float32),
    )(x, P)


def _layer2_body(pa_ref, pb_ref, p2_ref, rw_ref, rb_ref, z_ref, r_ref, h_ref):
    t = pl.program_id(1)

    @pl.when(t == 0)
    def _():
        xa = pa_ref[0] + pa_ref[1]
        xb = pb_ref[0] + pb_ref[1]
        h_ref[:, :WACC] = jnp.where(xa > 0, xa, jnp.exp(xa) - 1.0)
        h_ref[:, WACC:] = jnp.where(xb > 0, xb, jnp.exp(xb) - 1.0)
        r_ref[...] = (
            jnp.dot(h_ref[...], rw_ref[...], preferred_element_type=jnp.float32)
            + rb_ref[...]
        )

    z_ref[0] = jnp.dot(h_ref[...], p2_ref[0], preferred_element_type=jnp.float32)


def _layer2_tables(pa, pb, P2, rw, rb):
    return pl.pallas_call(
        _layer2_body,
        grid=(N // BN, T),
        in_specs=[
            pl.BlockSpec((2, BN, WACC), lambda i, t: (0, i, 0)),
            pl.BlockSpec((2, BN, WACC), lambda i, t: (0, i, 0)),
            pl.BlockSpec((1, D_IN, DROW), lambda i, t: (t, 0, 0)),
            pl.BlockSpec((D_IN, OUT), lambda i, t: (0, 0)),
            pl.BlockSpec((1, OUT), lambda i, t: (0, 0)),
        ],
        out_specs=[
            pl.BlockSpec((1, BN, DROW), lambda i, t: (t, i, 0)),
            pl.BlockSpec((BN, OUT), lambda i, t: (i, 0)),
        ],
        out_shape=[
            jax.ShapeDtypeStruct((T, N, DROW), jnp.float32),
            jax.ShapeDtypeStruct((N, OUT), jnp.float32),
        ],
        scratch_shapes=[pltpu.VMEM((BN, D_IN), jnp.float32)],
    )(pa, pb, P2, rw, rb)


def _final_body(q_ref, r_ref, o_ref):
    o_ref[...] = q_ref[0] + q_ref[1] + r_ref[...]


def _final_combine(q, R):
    return pl.pallas_call(
        _final_body,
        grid=(N // BN,),
        in_specs=[
            pl.BlockSpec((2, BN, OUT), lambda i: (0, i, 0)),
            pl.BlockSpec((BN, OUT), lambda i: (i, 0)),
        ],
        out_specs=pl.BlockSpec((BN, OUT), lambda i: (i, 0)),
        out_shape=jax.ShapeDtypeStruct((N, OUT), jnp.float32),
    )(q, R)


def _make_sc_edge_kernel(D, W, NH, CHK=CH):
    mesh = plsc.VectorSubcoreMesh(core_axis_name="c", subcore_axis_name="s")

    @functools.partial(
        pl.kernel,
        mesh=mesh,
        out_type=jax.ShapeDtypeStruct((NC * NP, W), jnp.float32),
        compiler_params=pltpu.CompilerParams(use_tc_tiling_on_sc=False),
        scratch_types=[
            pltpu.VMEM((KMAX * CHK,), jnp.int32),
            pltpu.VMEM((KMAX * CHK,), jnp.float32),
            pltpu.VMEM((CHK,), jnp.int32),
            pltpu.VMEM((CHK,), jnp.int32),
            pltpu.VMEM((CHK, D), jnp.float32),
            pltpu.VMEM((CHK, D), jnp.float32),
            pltpu.VMEM((CHK, W), jnp.float32),
            pltpu.VMEM((CHK, W), jnp.float32),
            pltpu.VMEM_SHARED((NP, W), jnp.float32),
            pltpu.SemaphoreType.DMA,
            pltpu.SemaphoreType.DMA,
            pltpu.SemaphoreType.DMA,
            pltpu.SemaphoreType.DMA,
            pltpu.SemaphoreType.DMA,
            pltpu.SemaphoreType.DMA,
        ],
    )
    def k(table_h, si_h, dst_h, ew_h, zero_h, out_h,
          si_t, ew_t, dst_a, dst_b, rows_a, rows_b, al_a, al_b, acc_sh,
          gsem_a, gsem_b, dsem_a, dsem_b, ssem_a, ssem_b):
        cid = lax.axis_index("c")
        sid = lax.axis_index("s")
        r0 = pl.multiple_of(sid * ROWS_PER_TILE, 8)
        pltpu.sync_copy(zero_h.at[pl.ds(r0, ROWS_PER_TILE)],
                        acc_sh.at[pl.ds(r0, ROWS_PER_TILE)])

        @pl.when(sid == NS - 1)
        def _():
            pltpu.sync_copy(
                zero_h.at[pl.ds(NS * ROWS_PER_TILE, NP - NS * ROWS_PER_TILE)],
                acc_sh.at[pl.ds(NS * ROWS_PER_TILE, NP - NS * ROWS_PER_TILE)])

        plsc.subcore_barrier()

        nch = jnp.where(cid == 0, K0, K1)
        base0 = sid * EPG + jnp.where(cid == 0, 0, K0 * CHK)
        base0 = pl.multiple_of(base0, 8)

        pltpu.sync_copy(si_h.at[pl.ds(base0, KMAX * CHK)], si_t)
        pltpu.sync_copy(ew_h.at[pl.ds(base0, KMAX * CHK)], ew_t)
        pltpu.async_copy(table_h.at[si_t.at[pl.ds(0, CHK)]], rows_a, gsem_a)
        pltpu.async_copy(table_h.at[si_t.at[pl.ds(CHK, CHK)]], rows_b, gsem_b)

        def process(c, dst_v, rows_v, al_v, gsem, dsem, ssem):
            off = pl.multiple_of(c * CHK, 8)
            base = pl.multiple_of(base0 + c * CHK, 8)

            @pl.when(c >= 2)
            def _():
                pltpu.make_async_copy(al_v, acc_sh.at[dst_v], ssem).wait()

            pltpu.async_copy(dst_h.at[pl.ds(base, CHK)], dst_v, dsem)
            pltpu.make_async_copy(
                table_h.at[si_t.at[pl.ds(off, CHK)]], rows_v, gsem).wait()

            def group(g, carry2):
                ew16 = ew_t[pl.ds(off + g * 16, 16)]
                for j in range(16):
                    e = g * 16 + j
                    lv = rows_v[e, pl.ds(W, 16)]
                    lv = jnp.where(lv >= 0, lv, SLOPE * lv)
                    att = (1.0 / (1.0 + jnp.exp(-lv))) * ew16[j]
                    for v in range(W // 16):
                        hk = (v * NH * 16) // W
                        al_v[e, pl.ds(v * 16, 16)] = (
                            rows_v[e, pl.ds(v * 16, 16)] * att[hk]
                        )
                return carry2

            lax.fori_loop(0, CHK // 16, group, 0)

            @pl.when(c + 2 < nch)
            def _():
                noff = pl.multiple_of(off + 2 * CHK, 8)
                pltpu.async_copy(
                    table_h.at[si_t.at[pl.ds(noff, CHK)]], rows_v, gsem)

            pltpu.make_async_copy(dst_h.at[pl.ds(base, CHK)], dst_v, dsem).wait()
            pltpu.async_copy(al_v, acc_sh.at[dst_v], ssem, add=True)

        def pair(p, carry):
            process(2 * p, dst_a, rows_a, al_a, gsem_a, dsem_a, ssem_a)
            process(2 * p + 1, dst_b, rows_b, al_b, gsem_b, dsem_b, ssem_b)
            return carry

        lax.fori_loop(0, nch // 2, pair, 0)
        pltpu.make_async_copy(al_a, acc_sh.at[dst_a], ssem_a).wait()
        pltpu.make_async_copy(al_b, acc_sh.at[dst_b], ssem_b).wait()
        plsc.subcore_barrier()
        pltpu.sync_copy(acc_sh.at[pl.ds(r0, ROWS_PER_TILE)],
                        out_h.at[pl.ds(cid * NP + r0, ROWS_PER_TILE)])

        @pl.when(sid == NS - 1)
        def _():
            pltpu.sync_copy(
                acc_sh.at[pl.ds(NS * ROWS_PER_TILE, NP - NS * ROWS_PER_TILE)],
                out_h.at[pl.ds(cid * NP + NS * ROWS_PER_TILE,
                               NP - NS * ROWS_PER_TILE)])

    return k


_sc_edge = _make_sc_edge_kernel(DROW, WACC, 4)


def kernel(feat, edge_index, edge_weight, ntype_idxs, etype_idxs,
           W1, a_l1, a_r1, W2, a_l2, a_r2, res_W2, res_b2):
    src = edge_index[0]
    dst = edge_index[1]

    a1 = (a_l1 + a_r1).reshape(T, D_IN, HEADS1, HID).sum(-1)
    C1 = jnp.matmul(W1, a1)
    zpad = jnp.zeros((T, D_IN, DROW - WACC - 4), jnp.float32)
    P1A = jnp.concatenate([W1[:, :, :WACC], C1[:, :, :4], zpad], axis=2)
    P1B = jnp.concatenate([W1[:, :, WACC:], C1[:, :, 4:], zpad], axis=2)

    idxc = jnp.arange(D_IN)
    perm = (idxc % HID) * HEADS1 + idxc // HID
    W2p = W2[:, perm, :]
    rwp = res_W2[perm, :]
    a2 = (a_l2 + a_r2).sum(axis=2)
    C2 = jnp.einsum('tko,to->tk', W2p, a2)
    P2 = jnp.concatenate(
        [W2p, jnp.repeat(C2[:, :, None], 4, axis=2), zpad], axis=2)

    pad = EPALLOC - E
    si = jnp.concatenate([etype_idxs * N + src,
                          jnp.zeros((pad,), jnp.int32)])
    dstp = jnp.concatenate([dst, jnp.zeros((pad,), jnp.int32)])
    ewp = jnp.concatenate([edge_weight, jnp.zeros((pad,), jnp.float32)])

    Z1A = _build_tables(feat, P1A).reshape(T * N, DROW)
    Z1B = _build_tables(feat, P1B).reshape(T * N, DROW)
    zeros = jnp.zeros((NP, WACC), jnp.float32)
    pa = _sc_edge(Z1A, si, dstp, ewp, zeros).reshape(NC, NP, WACC)
    pb = _sc_edge(Z1B, si, dstp, ewp, zeros).reshape(NC, NP, WACC)

    Z2_R = _layer2_tables(pa, pb, P2, rwp, res_b2.reshape(1, OUT))
    Z2 = Z2_R[0].reshape(T * N, DROW)
    R = Z2_R[1]

    q = _sc_edge(Z2, si, dstp, ewp, zeros).reshape(NC, NP, OUT)
    return _final_combine(q, R)


# speedup vs baseline: 1.7240x; 1.0182x over previous
import functools

import jax
import jax.numpy as jnp
from jax import lax
from jax.experimental import pallas as pl
from jax.experimental.pallas import tpu as pltpu
from jax.experimental.pallas import tpu_sc as plsc

N = 10000
E = 160000
T = 4
D_IN = 128
HEADS1 = 8
HID = 16
OUT = 64
SLOPE = 0.2

NC = 2
NS = 16
NWK = NC * NS
CH = 128
EP = 163840
EPG = EP // NS
K0 = 42
K1 = 38
KMAX = 42
EPALLOC = EP + (KMAX - K1) * CH
NP = N
ROWS_PER_TILE = 624

DROW = 80
WACC = 64

BN = 1000


def _tables_body(x_ref, p_ref, z_ref):
    z_ref[0] = jnp.dot(x_ref[...], p_ref[0], preferred_element_type=jnp.float32)


def _build_tables(x, P):
    T_, K, Do = P.shape
    return pl.pallas_call(
        _tables_body,
        grid=(T_, N // BN),
        in_specs=[
            pl.BlockSpec((BN, K), lambda t, i: (i, 0)),
            pl.BlockSpec((1, K, Do), lambda t, i: (t, 0, 0)),
        ],
        out_specs=pl.BlockSpec((1, BN, Do), lambda t, i: (t, i, 0)),
        out_shape=jax.ShapeDtypeStruct((T_, N, Do), jnp.float32),
    )(x, P)


def _layer2_body(pa_ref, pb_ref, p2_ref, rw_ref, rb_ref, z_ref, r_ref, h_ref):
    t = pl.program_id(1)

    @pl.when(t == 0)
    def _():
        xa = pa_ref[0] + pa_ref[1]
        xb = pb_ref[0] + pb_ref[1]
        h_ref[:, :WACC] = jnp.where(xa > 0, xa, jnp.exp(xa) - 1.0)
        h_ref[:, WACC:] = jnp.where(xb > 0, xb, jnp.exp(xb) - 1.0)
        r_ref[...] = (
            jnp.dot(h_ref[...], rw_ref[...], preferred_element_type=jnp.float32)
            + rb_ref[...]
        )

    z_ref[0] = jnp.dot(h_ref[...], p2_ref[0], preferred_element_type=jnp.float32)


def _layer2_tables(pa, pb, P2, rw, rb):
    return pl.pallas_call(
        _layer2_body,
        grid=(N // BN, T),
        in_specs=[
            pl.BlockSpec((2, BN, WACC), lambda i, t: (0, i, 0)),
            pl.BlockSpec((2, BN, WACC), lambda i, t: (0, i, 0)),
            pl.BlockSpec((1, D_IN, DROW), lambda i, t: (t, 0, 0)),
            pl.BlockSpec((D_IN, OUT), lambda i, t: (0, 0)),
            pl.BlockSpec((1, OUT), lambda i, t: (0, 0)),
        ],
        out_specs=[
            pl.BlockSpec((1, BN, DROW), lambda i, t: (t, i, 0)),
            pl.BlockSpec((BN, OUT), lambda i, t: (i, 0)),
        ],
        out_shape=[
            jax.ShapeDtypeStruct((T, N, DROW), jnp.float32),
            jax.ShapeDtypeStruct((N, OUT), jnp.float32),
        ],
        scratch_shapes=[pltpu.VMEM((BN, D_IN), jnp.float32)],
    )(pa, pb, P2, rw, rb)


def _final_body(q_ref, r_ref, o_ref):
    o_ref[...] = q_ref[0] + q_ref[1] + r_ref[...]


def _final_combine(q, R):
    return pl.pallas_call(
        _final_body,
        grid=(N // BN,),
        in_specs=[
            pl.BlockSpec((2, BN, OUT), lambda i: (0, i, 0)),
            pl.BlockSpec((BN, OUT), lambda i: (i, 0)),
        ],
        out_specs=pl.BlockSpec((BN, OUT), lambda i: (i, 0)),
        out_shape=jax.ShapeDtypeStruct((N, OUT), jnp.float32),
    )(q, R)


def _make_sc_edge_kernel(D, W, NH, CHK=CH):
    mesh = plsc.VectorSubcoreMesh(core_axis_name="c", subcore_axis_name="s")

    @functools.partial(
        pl.kernel,
        mesh=mesh,
        out_type=jax.ShapeDtypeStruct((NC * NP, W), jnp.float32),
        compiler_params=pltpu.CompilerParams(use_tc_tiling_on_sc=False),
        scratch_types=[
            pltpu.VMEM((KMAX * CHK,), jnp.int32),
            pltpu.VMEM((KMAX * CHK,), jnp.float32),
            pltpu.VMEM((CHK,), jnp.int32),
            pltpu.VMEM((CHK,), jnp.int32),
            pltpu.VMEM((CHK, D), jnp.float32),
            pltpu.VMEM((CHK, D), jnp.float32),
            pltpu.VMEM((CHK, W), jnp.float32),
            pltpu.VMEM((CHK, W), jnp.float32),
            pltpu.VMEM_SHARED((NP, W), jnp.float32),
            pltpu.SemaphoreType.DMA,
            pltpu.SemaphoreType.DMA,
            pltpu.SemaphoreType.DMA,
            pltpu.SemaphoreType.DMA,
            pltpu.SemaphoreType.DMA,
            pltpu.SemaphoreType.DMA,
        ],
    )
    def k(table_h, si_h, dst_h, ew_h, zero_h, out_h,
          si_t, ew_t, dst_a, dst_b, rows_a, rows_b, al_a, al_b, acc_sh,
          gsem_a, gsem_b, dsem_a, dsem_b, ssem_a, ssem_b):
        cid = lax.axis_index("c")
        sid = lax.axis_index("s")
        r0 = pl.multiple_of(sid * ROWS_PER_TILE, 8)
        pltpu.sync_copy(zero_h.at[pl.ds(r0, ROWS_PER_TILE)],
                        acc_sh.at[pl.ds(r0, ROWS_PER_TILE)])

        @pl.when(sid == NS - 1)
        def _():
            pltpu.sync_copy(
                zero_h.at[pl.ds(NS * ROWS_PER_TILE, NP - NS * ROWS_PER_TILE)],
                acc_sh.at[pl.ds(NS * ROWS_PER_TILE, NP - NS * ROWS_PER_TILE)])

        plsc.subcore_barrier()

        nch = jnp.where(cid == 0, K0, K1)
        base0 = sid * EPG + jnp.where(cid == 0, 0, K0 * CHK)
        base0 = pl.multiple_of(base0, 8)

        pltpu.sync_copy(si_h.at[pl.ds(base0, KMAX * CHK)], si_t)
        pltpu.sync_copy(ew_h.at[pl.ds(base0, KMAX * CHK)], ew_t)
        pltpu.async_copy(table_h.at[si_t.at[pl.ds(0, CHK)]], rows_a, gsem_a)
        pltpu.async_copy(table_h.at[si_t.at[pl.ds(CHK, CHK)]], rows_b, gsem_b)

        def process(c, dst_v, rows_v, al_v, gsem, dsem, ssem):
            off = pl.multiple_of(c * CHK, 8)
            base = pl.multiple_of(base0 + c * CHK, 8)

            @pl.when(c >= 2)
            def _():
                pltpu.make_async_copy(al_v, acc_sh.at[dst_v], ssem).wait()

            pltpu.async_copy(dst_h.at[pl.ds(base, CHK)], dst_v, dsem)
            pltpu.make_async_copy(
                table_h.at[si_t.at[pl.ds(off, CHK)]], rows_v, gsem).wait()

            def group(g, carry2):
                ew16 = ew_t[pl.ds(off + g * 16, 16)]
                for j in range(16):
                    e = g * 16 + j
                    lv = rows_v[e, pl.ds(W, 16)]
                    lv = jnp.where(lv >= 0, lv, SLOPE * lv)
                    att = (1.0 / (1.0 + jnp.exp(-lv))) * ew16[j]
                    for v in range(W // 16):
                        hk = (v * NH * 16) // W
                        al_v[e, pl.ds(v * 16, 16)] = (
                            rows_v[e, pl.ds(v * 16, 16)] * att[hk]
                        )
                return carry2

            lax.fori_loop(0, CHK // 16, group, 0)

            @pl.when(c + 2 < nch)
            def _():
                noff = pl.multiple_of(off + 2 * CHK, 8)
                pltpu.async_copy(
                    table_h.at[si_t.at[pl.ds(noff, CHK)]], rows_v, gsem)

            pltpu.make_async_copy(dst_h.at[pl.ds(base, CHK)], dst_v, dsem).wait()
            pltpu.async_copy(al_v, acc_sh.at[dst_v], ssem, add=True)

        def pair(p, carry):
            process(2 * p, dst_a, rows_a, al_a, gsem_a, dsem_a, ssem_a)
            process(2 * p + 1, dst_b, rows_b, al_b, gsem_b, dsem_b, ssem_b)
            return carry

        lax.fori_loop(0, nch // 2, pair, 0)
        pltpu.make_async_copy(al_a, acc_sh.at[dst_a], ssem_a).wait()
        pltpu.make_async_copy(al_b, acc_sh.at[dst_b], ssem_b).wait()
        plsc.subcore_barrier()
        pltpu.sync_copy(acc_sh.at[pl.ds(r0, ROWS_PER_TILE)],
                        out_h.at[pl.ds(cid * NP + r0, ROWS_PER_TILE)])

        @pl.when(sid == NS - 1)
        def _():
            pltpu.sync_copy(
                acc_sh.at[pl.ds(NS * ROWS_PER_TILE, NP - NS * ROWS_PER_TILE)],
                out_h.at[pl.ds(cid * NP + NS * ROWS_PER_TILE,
                               NP - NS * ROWS_PER_TILE)])

    return k


_sc_edge = _make_sc_edge_kernel(DROW, WACC, 4)


def kernel(feat, edge_index, edge_weight, ntype_idxs, etype_idxs,
           W1, a_l1, a_r1, W2, a_l2, a_r2, res_W2, res_b2):
    src = edge_index[0]
    dst = edge_index[1]

    a1 = (a_l1 + a_r1).reshape(T, D_IN, HEADS1, HID).sum(-1)
    C1 = jnp.matmul(W1, a1)
    zpad = jnp.zeros((T, D_IN, DROW - WACC - 4), jnp.float32)
    P1A = jnp.concatenate([W1[:, :, :WACC], C1[:, :, :4], zpad], axis=2)
    P1B = jnp.concatenate([W1[:, :, WACC:], C1[:, :, 4:], zpad], axis=2)

    idxc = jnp.arange(D_IN)
    perm = (idxc % HID) * HEADS1 + idxc // HID
    W2p = W2[:, perm, :]
    rwp = res_W2[perm, :]
    a2 = (a_l2 + a_r2).sum(axis=2)
    C2 = jnp.einsum('tko,to->tk', W2p, a2)
    P2 = jnp.concatenate(
        [W2p, jnp.repeat(C2[:, :, None], 4, axis=2), zpad], axis=2)

    pad = EPALLOC - E
    si = jnp.concatenate([etype_idxs * N + src,
                          jnp.zeros((pad,), jnp.int32)])
    dstp = jnp.concatenate([dst, jnp.zeros((pad,), jnp.int32)])
    ewp = jnp.concatenate([edge_weight, jnp.zeros((pad,), jnp.float32)])

    Z1 = _build_tables(feat, jnp.concatenate([P1A, P1B], axis=0))
    Z1 = Z1.reshape(2 * T * N, DROW)
    zeros = jnp.zeros((NP, WACC), jnp.float32)
    pa = _sc_edge(Z1, si, dstp, ewp, zeros).reshape(NC, NP, WACC)
    pb = _sc_edge(Z1, si + T * N, dstp, ewp, zeros).reshape(NC, NP, WACC)

    Z2_R = _layer2_tables(pa, pb, P2, rwp, res_b2.reshape(1, OUT))
    Z2 = Z2_R[0].reshape(T * N, DROW)
    R = Z2_R[1]

    q = _sc_edge(Z2, si, dstp, ewp, zeros).reshape(NC, NP, OUT)
    return _final_combine(q, R)
